# Initial kernel scaffold; baseline (speedup 1.0000x reference)
import jax
import jax.numpy as jnp
from jax.experimental import pallas as pl


def kernel(logits, targets):
    raise NotImplementedError("write your pallas kernel here")


# speedup vs baseline: 11.1316x; 11.1316x over previous
import functools

import jax
import jax.numpy as jnp
from jax.experimental import pallas as pl
from jax.experimental.pallas import tpu as pltpu

SMOOTH = 1.0
IGNORE_INDEX = 0


def _dice_body(x_ref, t_ref, out_ref, acc_ref, *, B, C, nchunk):
    b = pl.program_id(0)
    n = pl.program_id(1)

    @pl.when((b == 0) & (n == 0))
    def _init():
        acc_ref[...] = jnp.zeros_like(acc_ref)

    x = x_ref[0]
    t = t_ref[0, 0]

    m = jnp.max(x, axis=0, keepdims=True)
    e = jnp.exp(x - m)
    s = jnp.sum(e, axis=0, keepdims=True)
    p = e * (1.0 / s)

    cls = jax.lax.broadcasted_iota(jnp.int32, (C, 1), 0)
    mask = t[None, :] == cls

    inter = jnp.sum(jnp.where(mask, p, 0.0), axis=1)
    psum = jnp.sum(p, axis=1)
    cnt = jnp.sum(mask.astype(jnp.float32), axis=1)

    acc_ref[b, 0, :] += inter
    acc_ref[b, 1, :] += psum
    acc_ref[b, 2, :] += cnt

    @pl.when((b == B - 1) & (n == nchunk - 1))
    def _finish():
        inter_bc = acc_ref[:, 0, :]
        union_bc = acc_ref[:, 1, :] + acc_ref[:, 2, :]
        dice = (2.0 * inter_bc + SMOOTH) / (union_bc + SMOOTH)
        cmask = (jax.lax.broadcasted_iota(jnp.int32, (1, C), 1)
                 != IGNORE_INDEX).astype(jnp.float32)
        mean_dice = jnp.sum(dice * cmask) / (B * (C - 1))
        out_ref[0] = 1.0 - mean_dice


def kernel(logits, targets):
    B, C = logits.shape[0], logits.shape[1]
    N = targets.shape[1] * targets.shape[2] * targets.shape[3]
    x = logits.astype(jnp.float32).reshape(B, C, N)
    t = targets.reshape(B, 1, N)

    TN = min(131072, N)
    nchunk = N // TN

    body = functools.partial(_dice_body, B=B, C=C, nchunk=nchunk)
    out = pl.pallas_call(
        body,
        grid=(B, nchunk),
        in_specs=[
            pl.BlockSpec((1, C, TN), lambda b, n: (b, 0, n)),
            pl.BlockSpec((1, 1, TN), lambda b, n: (b, 0, n)),
        ],
        out_specs=pl.BlockSpec(memory_space=pltpu.SMEM),
        out_shape=jax.ShapeDtypeStruct((1,), jnp.float32),
        scratch_shapes=[pltpu.VMEM((B, 3, C), jnp.float32)],
    )(x, t)
    return out[0]
